# baseline (speedup 1.0000x reference)
import jax
import jax.numpy as jnp
from jax import lax
from jax.experimental import pallas as pl
from jax.experimental.pallas import tpu as pltpu
from jax.experimental.pallas import tpu_sc as plsc

B = 4096
L = 200
D = 128
LANES = 16
NGRP = D // LANES

CH = 128

NC = 2
NS = 16
NW = NC * NS
SPW = B // NW
NCH = SPW * L // CH
NBUF = 5

RUNROLL = 8


def _body(idx_hbm, bs_hbm, table_hbm, out_hbm,
          idx_v, bs_v, buf0, buf1, buf2, buf3, buf4, out_v,
          sem0, sem1, sem2, sem3, sem4):
    wid = lax.axis_index("s") * NC + lax.axis_index("c")
    seq0 = wid * SPW

    pltpu.sync_copy(idx_hbm.at[pl.ds(wid * NCH, NCH)], idx_v)
    pltpu.sync_copy(bs_hbm.at[pl.ds(seq0, SPW)], bs_v)

    bufs = (buf0, buf1, buf2, buf3, buf4)
    sems = (sem0, sem1, sem2, sem3, sem4)

    for c in range(NBUF):
        pltpu.async_copy(table_hbm.at[idx_v.at[c]], bufs[c], sems[c])

    def red_range(buf, base0, ntrips, acc):
        def red(i, a):
            base = base0 + i * RUNROLL
            new = []
            for g in range(NGRP):
                v = [buf[base + j, pl.ds(g * LANES, LANES)]
                     for j in range(RUNROLL)]
                t = ((v[0] + v[1]) + (v[2] + v[3])) + \
                    ((v[4] + v[5]) + (v[6] + v[7]))
                new.append(a[g] + t)
            return tuple(new)
        return lax.fori_loop(0, ntrips, red, acc)

    def finalize(s, acc):
        bs = plsc.load_gather(bs_v, [jnp.full((LANES,), s, jnp.int32)])
        scale = 1.0 / bs.astype(jnp.float32)
        for g in range(NGRP):
            out_v[s, pl.ds(g * LANES, LANES)] = acc[g] * scale

    zeros = tuple(jnp.zeros((LANES,), jnp.float32) for _ in range(NGRP))

    def grp_body(v, carry):
        r0, s_cur, acc = carry
        for jj in range(NBUF):
            c = NBUF * v + jj
            buf, sem = bufs[jj], sems[jj]
            pltpu.make_async_copy(table_hbm.at[idx_v.at[c]], buf, sem).wait()

            t = r0 + CH
            has_b = t >= L
            n1 = jnp.minimum(L - r0, CH)
            acc = red_range(buf, 0, n1 // RUNROLL, acc)

            @pl.when(has_b)
            def _():
                finalize(s_cur, acc)

            acc = tuple(jnp.where(has_b, z, a) for z, a in zip(zeros, acc))
            acc = red_range(buf, n1, (CH - n1) // RUNROLL, acc)
            s_cur = s_cur + has_b.astype(jnp.int32)
            r0 = jnp.where(has_b, t - L, t)

            @pl.when(c + NBUF < NCH)
            def _():
                pltpu.async_copy(table_hbm.at[idx_v.at[c + NBUF]], buf, sem)
        return r0, s_cur, acc

    lax.fori_loop(0, NCH // NBUF,
                  grp_body, (jnp.int32(0), jnp.int32(0), zeros))
    pltpu.sync_copy(out_v, out_hbm.at[pl.ds(seq0, SPW)])


@jax.jit
def _embed_bag(idx_rows, batch_sizes, table):
    mesh = plsc.VectorSubcoreMesh(core_axis_name="c", subcore_axis_name="s")
    return pl.kernel(
        _body,
        out_type=jax.ShapeDtypeStruct((B, D), jnp.float32),
        mesh=mesh,
        compiler_params=pltpu.CompilerParams(needs_layout_passes=False),
        scratch_types=[
            pltpu.VMEM((NCH, CH), jnp.int32),
            pltpu.VMEM((SPW,), jnp.int32),
            pltpu.VMEM((CH, D), jnp.float32),
            pltpu.VMEM((CH, D), jnp.float32),
            pltpu.VMEM((CH, D), jnp.float32),
            pltpu.VMEM((CH, D), jnp.float32),
            pltpu.VMEM((CH, D), jnp.float32),
            pltpu.VMEM((SPW, D), jnp.float32),
            pltpu.SemaphoreType.DMA,
            pltpu.SemaphoreType.DMA,
            pltpu.SemaphoreType.DMA,
            pltpu.SemaphoreType.DMA,
            pltpu.SemaphoreType.DMA,
        ],
    )(idx_rows, batch_sizes, table)


def kernel(word_inputs_data, batch_sizes, embedding_table):
    idx = word_inputs_data.astype(jnp.int32)
    return _embed_bag(idx.reshape(B * L // CH, CH),
                      batch_sizes.astype(jnp.int32), embedding_table)


# speedup vs baseline: 1.3342x; 1.3342x over previous
import jax
import jax.numpy as jnp
from jax import lax
from jax.experimental import pallas as pl
from jax.experimental.pallas import tpu as pltpu
from jax.experimental.pallas import tpu_sc as plsc

B = 4096
L = 200
D = 128
LANES = 16
NGRP = D // LANES

CHA = 128
CHB = 72

NC = 2
NS = 16
NW = NC * NS
SPW = B // NW

RUNROLL = 8


def _body(idxa_hbm, idxb_hbm, bs_hbm, table_hbm, out_hbm,
          idxa_v, idxb_v, bs_v, bufa0, bufb0, bufa1, bufb1, out_v,
          sema0, semb0, sema1, semb1):
    wid = lax.axis_index("s") * NC + lax.axis_index("c")
    seq0 = wid * SPW

    pltpu.sync_copy(idxa_hbm.at[pl.ds(seq0, SPW)], idxa_v)
    pltpu.sync_copy(idxb_hbm.at[pl.ds(seq0, SPW)], idxb_v)
    pltpu.sync_copy(bs_hbm.at[pl.ds(seq0, SPW)], bs_v)

    bufsa = (bufa0, bufa1)
    bufsb = (bufb0, bufb1)
    semsa = (sema0, sema1)
    semsb = (semb0, semb1)

    for h in range(2):
        pltpu.async_copy(table_hbm.at[idxa_v.at[h]], bufsa[h], semsa[h])
        pltpu.async_copy(table_hbm.at[idxb_v.at[h]], bufsb[h], semsb[h])

    def reduce_chunk(buf, nrows, acc):
        def red(i, a):
            base = i * RUNROLL
            new = []
            for g in range(NGRP):
                v = [buf[base + j, pl.ds(g * LANES, LANES)]
                     for j in range(RUNROLL)]
                t = ((v[0] + v[1]) + (v[2] + v[3])) + \
                    ((v[4] + v[5]) + (v[6] + v[7]))
                new.append(a[g] + t)
            return tuple(new)
        return lax.fori_loop(0, nrows // RUNROLL, red, acc)

    def blk_body(i, carry):
        for half in range(2):
            s = 2 * i + half
            acc = tuple(jnp.zeros((LANES,), jnp.float32)
                        for _ in range(NGRP))
            for part, (idx_v, bufs, sems, nrows) in enumerate((
                    (idxa_v, bufsa, semsa, CHA),
                    (idxb_v, bufsb, semsb, CHB))):
                buf, sem = bufs[half], sems[half]
                pltpu.make_async_copy(
                    table_hbm.at[idx_v.at[s]], buf, sem).wait()
                acc = reduce_chunk(buf, nrows, acc)

                @pl.when(s + 2 < SPW)
                def _():
                    pltpu.async_copy(
                        table_hbm.at[idx_v.at[s + 2]], buf, sem)

            bs = plsc.load_gather(bs_v, [jnp.full((LANES,), s, jnp.int32)])
            scale = 1.0 / bs.astype(jnp.float32)
            for g in range(NGRP):
                out_v[s, pl.ds(g * LANES, LANES)] = acc[g] * scale
        return carry

    lax.fori_loop(0, SPW // 2, blk_body, 0)
    pltpu.sync_copy(out_v, out_hbm.at[pl.ds(seq0, SPW)])


@jax.jit
def _embed_bag(idx_a, idx_b, batch_sizes, table):
    mesh = plsc.VectorSubcoreMesh(core_axis_name="c", subcore_axis_name="s")
    return pl.kernel(
        _body,
        out_type=jax.ShapeDtypeStruct((B, D), jnp.float32),
        mesh=mesh,
        compiler_params=pltpu.CompilerParams(needs_layout_passes=False),
        scratch_types=[
            pltpu.VMEM((SPW, CHA), jnp.int32),
            pltpu.VMEM((SPW, CHB), jnp.int32),
            pltpu.VMEM((SPW,), jnp.int32),
            pltpu.VMEM((CHA, D), jnp.float32),
            pltpu.VMEM((CHB, D), jnp.float32),
            pltpu.VMEM((CHA, D), jnp.float32),
            pltpu.VMEM((CHB, D), jnp.float32),
            pltpu.VMEM((SPW, D), jnp.float32),
            pltpu.SemaphoreType.DMA,
            pltpu.SemaphoreType.DMA,
            pltpu.SemaphoreType.DMA,
            pltpu.SemaphoreType.DMA,
        ],
    )(idx_a, idx_b, batch_sizes, table)


def kernel(word_inputs_data, batch_sizes, embedding_table):
    idx = word_inputs_data.astype(jnp.int32)
    return _embed_bag(idx[:, :CHA], idx[:, CHA:CHA + CHB],
                      batch_sizes.astype(jnp.int32), embedding_table)


# speedup vs baseline: 1.4273x; 1.0698x over previous
import jax
import jax.numpy as jnp
from jax import lax
from jax.experimental import pallas as pl
from jax.experimental.pallas import tpu as pltpu
from jax.experimental.pallas import tpu_sc as plsc

B = 4096
L = 200
D = 128
LANES = 16
NGRP = D // LANES

CHA = 104
CHB = 96

NC = 2
NS = 16
NW = NC * NS
SPW = B // NW

RUNROLL = 8


def _body(idxa_hbm, idxb_hbm, bs_hbm, table_hbm, out_hbm,
          idxa_v, idxb_v, bs_v, bufa0, bufb0, bufa1, bufb1, out_v,
          sema0, semb0, sema1, semb1):
    wid = lax.axis_index("s") * NC + lax.axis_index("c")
    seq0 = wid * SPW

    pltpu.sync_copy(idxa_hbm.at[pl.ds(seq0, SPW)], idxa_v)
    pltpu.sync_copy(idxb_hbm.at[pl.ds(seq0, SPW)], idxb_v)
    pltpu.sync_copy(bs_hbm.at[pl.ds(seq0, SPW)], bs_v)

    bufsa = (bufa0, bufa1)
    bufsb = (bufb0, bufb1)
    semsa = (sema0, sema1)
    semsb = (semb0, semb1)

    for h in range(2):
        pltpu.async_copy(table_hbm.at[idxa_v.at[h]], bufsa[h], semsa[h])
        pltpu.async_copy(table_hbm.at[idxb_v.at[h]], bufsb[h], semsb[h])

    def reduce_chunk(buf, nrows, acc):
        def red(i, a):
            base = i * RUNROLL
            new = []
            for g in range(NGRP):
                v = [buf[base + j, pl.ds(g * LANES, LANES)]
                     for j in range(RUNROLL)]
                t = ((v[0] + v[1]) + (v[2] + v[3])) + \
                    ((v[4] + v[5]) + (v[6] + v[7]))
                new.append(a[g] + t)
            return tuple(new)
        return lax.fori_loop(0, nrows // RUNROLL, red, acc)

    def blk_body(i, carry):
        for half in range(2):
            s = 2 * i + half
            acc = tuple(jnp.zeros((LANES,), jnp.float32)
                        for _ in range(NGRP))
            for part, (idx_v, bufs, sems, nrows) in enumerate((
                    (idxa_v, bufsa, semsa, CHA),
                    (idxb_v, bufsb, semsb, CHB))):
                buf, sem = bufs[half], sems[half]
                pltpu.make_async_copy(
                    table_hbm.at[idx_v.at[s]], buf, sem).wait()
                acc = reduce_chunk(buf, nrows, acc)

                @pl.when(s + 2 < SPW)
                def _():
                    pltpu.async_copy(
                        table_hbm.at[idx_v.at[s + 2]], buf, sem)

            bs = plsc.load_gather(bs_v, [jnp.full((LANES,), s, jnp.int32)])
            scale = 1.0 / bs.astype(jnp.float32)
            for g in range(NGRP):
                out_v[s, pl.ds(g * LANES, LANES)] = acc[g] * scale
        return carry

    lax.fori_loop(0, SPW // 2, blk_body, 0)
    pltpu.sync_copy(out_v, out_hbm.at[pl.ds(seq0, SPW)])


@jax.jit
def _embed_bag(idx_a, idx_b, batch_sizes, table):
    mesh = plsc.VectorSubcoreMesh(core_axis_name="c", subcore_axis_name="s")
    return pl.kernel(
        _body,
        out_type=jax.ShapeDtypeStruct((B, D), jnp.float32),
        mesh=mesh,
        compiler_params=pltpu.CompilerParams(needs_layout_passes=False),
        scratch_types=[
            pltpu.VMEM((SPW, CHA), jnp.int32),
            pltpu.VMEM((SPW, CHB), jnp.int32),
            pltpu.VMEM((SPW,), jnp.int32),
            pltpu.VMEM((CHA, D), jnp.float32),
            pltpu.VMEM((CHB, D), jnp.float32),
            pltpu.VMEM((CHA, D), jnp.float32),
            pltpu.VMEM((CHB, D), jnp.float32),
            pltpu.VMEM((SPW, D), jnp.float32),
            pltpu.SemaphoreType.DMA,
            pltpu.SemaphoreType.DMA,
            pltpu.SemaphoreType.DMA,
            pltpu.SemaphoreType.DMA,
        ],
    )(idx_a, idx_b, batch_sizes, table)


def kernel(word_inputs_data, batch_sizes, embedding_table):
    idx = word_inputs_data.astype(jnp.int32)
    return _embed_bag(idx[:, :CHA], idx[:, CHA:],
                      batch_sizes.astype(jnp.int32), embedding_table)


# speedup vs baseline: 1.4500x; 1.0159x over previous
import jax
import jax.numpy as jnp
from jax import lax
from jax.experimental import pallas as pl
from jax.experimental.pallas import tpu as pltpu
from jax.experimental.pallas import tpu_sc as plsc

B = 4096
L = 200
D = 128
LANES = 16
NGRP = D // LANES

CHA = 104
CHB = 96

NC = 2
NS = 16
NW = NC * NS
SPW = B // NW

RUNROLL = 8


DEPTH = 3


def _body(idxa_hbm, idxb_hbm, bs_hbm, table_hbm, out_hbm,
          idxa_v, idxb_v, bs_v, bufa0, bufb0, bufa1, bufb1, bufa2, bufb2,
          out_v, sema0, semb0, sema1, semb1, sema2, semb2):
    wid = lax.axis_index("s") * NC + lax.axis_index("c")
    seq0 = wid * SPW

    pltpu.sync_copy(idxa_hbm.at[pl.ds(seq0, SPW)], idxa_v)
    pltpu.sync_copy(idxb_hbm.at[pl.ds(seq0, SPW)], idxb_v)
    pltpu.sync_copy(bs_hbm.at[pl.ds(seq0, SPW)], bs_v)

    bufsa = (bufa0, bufa1, bufa2)
    bufsb = (bufb0, bufb1, bufb2)
    semsa = (sema0, sema1, sema2)
    semsb = (semb0, semb1, semb2)

    for h in range(DEPTH):
        pltpu.async_copy(table_hbm.at[idxa_v.at[h]], bufsa[h], semsa[h])
        pltpu.async_copy(table_hbm.at[idxb_v.at[h]], bufsb[h], semsb[h])

    def reduce_chunk(buf, nrows, acc):
        def red(i, a):
            base = i * RUNROLL
            new = []
            for g in range(NGRP):
                v = [buf[base + j, pl.ds(g * LANES, LANES)]
                     for j in range(RUNROLL)]
                t = ((v[0] + v[1]) + (v[2] + v[3])) + \
                    ((v[4] + v[5]) + (v[6] + v[7]))
                new.append(a[g] + t)
            return tuple(new)
        return lax.fori_loop(0, nrows // RUNROLL, red, acc)

    def do_seq(s, half, refill):
        acc = tuple(jnp.zeros((LANES,), jnp.float32)
                    for _ in range(NGRP))
        for idx_v, bufs, sems, nrows in (
                (idxa_v, bufsa, semsa, CHA),
                (idxb_v, bufsb, semsb, CHB)):
            buf, sem = bufs[half], sems[half]
            pltpu.make_async_copy(
                table_hbm.at[idx_v.at[s]], buf, sem).wait()
            acc = reduce_chunk(buf, nrows, acc)

            if refill:
                @pl.when(s + DEPTH < SPW)
                def _():
                    pltpu.async_copy(
                        table_hbm.at[idx_v.at[s + DEPTH]], buf, sem)

        bs = plsc.load_gather(bs_v, [jnp.full((LANES,), s, jnp.int32)])
        scale = 1.0 / bs.astype(jnp.float32)
        for g in range(NGRP):
            out_v[s, pl.ds(g * LANES, LANES)] = acc[g] * scale

    def blk_body(i, carry):
        for half in range(DEPTH):
            do_seq(DEPTH * i + half, half, True)
        return carry

    nblk = SPW // DEPTH
    lax.fori_loop(0, nblk, blk_body, 0)
    for s in range(nblk * DEPTH, SPW):
        do_seq(jnp.int32(s), s % DEPTH, False)
    pltpu.sync_copy(out_v, out_hbm.at[pl.ds(seq0, SPW)])


@jax.jit
def _embed_bag(idx_a, idx_b, batch_sizes, table):
    mesh = plsc.VectorSubcoreMesh(core_axis_name="c", subcore_axis_name="s")
    return pl.kernel(
        _body,
        out_type=jax.ShapeDtypeStruct((B, D), jnp.float32),
        mesh=mesh,
        compiler_params=pltpu.CompilerParams(needs_layout_passes=False),
        scratch_types=[
            pltpu.VMEM((SPW, CHA), jnp.int32),
            pltpu.VMEM((SPW, CHB), jnp.int32),
            pltpu.VMEM((SPW,), jnp.int32),
            pltpu.VMEM((CHA, D), jnp.float32),
            pltpu.VMEM((CHB, D), jnp.float32),
            pltpu.VMEM((CHA, D), jnp.float32),
            pltpu.VMEM((CHB, D), jnp.float32),
            pltpu.VMEM((CHA, D), jnp.float32),
            pltpu.VMEM((CHB, D), jnp.float32),
            pltpu.VMEM((SPW, D), jnp.float32),
            pltpu.SemaphoreType.DMA,
            pltpu.SemaphoreType.DMA,
            pltpu.SemaphoreType.DMA,
            pltpu.SemaphoreType.DMA,
            pltpu.SemaphoreType.DMA,
            pltpu.SemaphoreType.DMA,
        ],
    )(idx_a, idx_b, batch_sizes, table)


def kernel(word_inputs_data, batch_sizes, embedding_table):
    idx = word_inputs_data.astype(jnp.int32)
    return _embed_bag(idx[:, :CHA], idx[:, CHA:],
                      batch_sizes.astype(jnp.int32), embedding_table)
